# baseline (speedup 1.0000x reference)
import jax
import jax.numpy as jnp
from jax.experimental import pallas as pl
from jax.experimental.pallas import tpu as pltpu

_LANE = 128
_TH = 7


def _shift_cols_right(a):
    zero = jnp.zeros_like(a[:, :1, :])
    return jnp.concatenate([zero, a[:, :-1, :]], axis=1)


def _shift_cols_left(a):
    zero = jnp.zeros_like(a[:, :1, :])
    return jnp.concatenate([a[:, 1:, :], zero], axis=1)


def _rb_kernel(x_ref, w1_ref, w2_ref, b1_ref, b2_ref, out_ref,
               xlc_ref, xr2_ref, hlc_ref, hr2_ref):
    H, W, C = x_ref.shape
    M = H * W
    MS = _TH * W
    MD = MS + W
    NP = H // (4 * _TH)

    zrow = jnp.zeros((1, W, 2 * C), jnp.bfloat16)
    for ref in (xlc_ref, xr2_ref, hlc_ref, hr2_ref):
        ref[0:1] = zrow
        ref[H + 1:H + 2] = zrow
    xr2_ref[H:H + 1, :, C:2 * C] = zrow[:, :, 0:C]
    hr2_ref[H:H + 1, :, C:2 * C] = zrow[:, :, 0:C]

    def _fill(lc_ref, r2_ref, v):
        lc_ref[1:H + 1, :, 0:C] = _shift_cols_right(v)
        lc_ref[1:H + 1, :, C:2 * C] = v
        vr = _shift_cols_left(v)
        r2_ref[1:H + 1, :, 0:C] = vr
        r2_ref[0:H, :, C:2 * C] = vr

    def _conv(lc_ref, r2_ref, w_ref, extra, emit):
        lc = lc_ref[...].reshape((H + 2) * W, 2 * C)
        r2 = r2_ref[...].reshape((H + 2) * W, 2 * C)
        w0, w1, w2 = w_ref[0], w_ref[1], w_ref[2]

        def strip(pair, mxu, half):
            return 4 * pair + 2 * mxu + half

        for pair in range(NP):
            for wk, sr in ((w0, 0), (w1, 1), (w2, 0)):
                for mxu in range(2):
                    pltpu.matmul_push_rhs(wk, staging_register=sr,
                                          mxu_index=mxu)
                for half in range(2):
                    for mxu in range(2):
                        s0 = strip(pair, mxu, half) * MS
                        if sr == 1 and wk is w1:
                            lhs = lc[s0 + W:s0 + W + MD]
                        elif wk is w2:
                            lhs = r2[s0:s0 + MD]
                        else:
                            lhs = lc[s0:s0 + MD]
                        pltpu.matmul_acc_lhs(
                            acc_addr=half * (MD // 4),
                            lhs=lhs, mxu_index=mxu,
                            load_staged_rhs=sr if half == 0 else None)
            for half in range(2):
                for mxu in range(2):
                    s = strip(pair, mxu, half)
                    r = pltpu.matmul_pop(
                        acc_addr=half * (MD // 4), shape=(MD, 2 * C),
                        dtype=jnp.float32, mxu_index=mxu)
                    s0 = s * MS
                    y = (r[0:MS, 0:C] + r[W:MD, C:2 * C]
                         + extra[s0:s0 + MS])
                    emit(s, y)

    _fill(xlc_ref, xr2_ref, x_ref[...].astype(jnp.bfloat16))

    def emit1(s, y):
        h = jnp.maximum(y, 0.0).reshape(_TH, W, C).astype(jnp.bfloat16)
        r0 = s * _TH
        hlc_ref[1 + r0:1 + r0 + _TH, :, 0:C] = _shift_cols_right(h)
        hlc_ref[1 + r0:1 + r0 + _TH, :, C:2 * C] = h
        hr = _shift_cols_left(h)
        hr2_ref[1 + r0:1 + r0 + _TH, :, 0:C] = hr
        hr2_ref[r0:r0 + _TH, :, C:2 * C] = hr

    b1 = b1_ref[...]
    _conv(xlc_ref, xr2_ref, w1_ref,
          jnp.broadcast_to(b1, (M, C)), emit1)

    extra2 = b2_ref[...] + x_ref[...].astype(jnp.float32).reshape(M, C)

    def emit2(s, y):
        r0 = s * _TH
        out_ref[r0:r0 + _TH] = jnp.maximum(y, 0.0).reshape(
            _TH, W, C).astype(out_ref.dtype)

    _conv(hlc_ref, hr2_ref, w2_ref, extra2, emit2)


def _pack_w4(w_hwio):
    w = w_hwio.astype(jnp.bfloat16)
    C = w.shape[2]
    z = jnp.zeros((C, C), jnp.bfloat16)
    r1 = jnp.block([[w[0, 0], w[1, 0]], [w[0, 1], w[1, 1]]])
    r2 = jnp.block([[z, w[2, 0]], [z, w[2, 1]]])
    r3 = jnp.block([[w[0, 2], w[1, 2]], [z, w[2, 2]]])
    return jnp.stack([r1, r2, r3])


def kernel(x_nhwc, w1f, bias1, w2f, bias2):
    N, H, W, C = x_nhwc.shape
    assert C % _LANE == 0 and W % 8 == 0 and H % (4 * _TH) == 0, \
        (N, H, W, C)

    w1c = _pack_w4(w1f)
    w2c = _pack_w4(w2f)
    b1 = bias1.astype(jnp.float32).reshape(1, C)
    b2 = bias2.astype(jnp.float32).reshape(1, C)

    def const_spec(shape):
        return pl.BlockSpec(shape, lambda n: tuple(0 for _ in shape),
                            pipeline_mode=pl.Buffered(1))

    return pl.pallas_call(
        _rb_kernel,
        out_shape=jax.ShapeDtypeStruct((N, H, W, C), x_nhwc.dtype),
        grid=(N,),
        in_specs=[
            pl.BlockSpec((None, H, W, C), lambda n: (n, 0, 0, 0)),
            const_spec((3, 2 * C, 2 * C)),
            const_spec((3, 2 * C, 2 * C)),
            const_spec((1, C)),
            const_spec((1, C)),
        ],
        out_specs=pl.BlockSpec((None, H, W, C), lambda n: (n, 0, 0, 0)),
        scratch_shapes=[pltpu.VMEM((H + 2, W, 2 * C), jnp.bfloat16)
                        for _ in range(4)],
        compiler_params=pltpu.CompilerParams(
            dimension_semantics=("parallel",)),
    )(x_nhwc, w1c, w2c, b1, b2)


# speedup vs baseline: 1.0586x; 1.0586x over previous
import jax
import jax.numpy as jnp
from jax.experimental import pallas as pl
from jax.experimental.pallas import tpu as pltpu

_LANE = 128
_TH = 7


def _shift_cols_right(a):
    zero = jnp.zeros_like(a[:, :1, :])
    return jnp.concatenate([zero, a[:, :-1, :]], axis=1)


def _shift_cols_left(a):
    zero = jnp.zeros_like(a[:, :1, :])
    return jnp.concatenate([a[:, 1:, :], zero], axis=1)


def _rb_kernel(x_ref, w1_ref, w2_ref, b1_ref, b2_ref, out_ref,
               xlc_ref, xr2_ref, hlc_ref, hr2_ref, w1s_ref, w2s_ref):
    H, W, C = x_ref.shape
    M = H * W
    MS = _TH * W
    MD = MS + W
    NP = H // (4 * _TH)

    def _pack():
        zc = jnp.zeros((C, C), jnp.bfloat16)
        for ws, w in ((w1s_ref, w1_ref), (w2s_ref, w2_ref)):
            ws[0, 0:C, 0:C] = w[0, 0]
            ws[0, 0:C, C:2 * C] = w[1, 0]
            ws[0, C:2 * C, 0:C] = w[0, 1]
            ws[0, C:2 * C, C:2 * C] = w[1, 1]
            ws[1, 0:C, 0:C] = zc
            ws[1, 0:C, C:2 * C] = w[2, 0]
            ws[1, C:2 * C, 0:C] = zc
            ws[1, C:2 * C, C:2 * C] = w[2, 1]
            ws[2, 0:C, 0:C] = w[0, 2]
            ws[2, 0:C, C:2 * C] = w[1, 2]
            ws[2, C:2 * C, 0:C] = zc
            ws[2, C:2 * C, C:2 * C] = w[2, 2]
    _pack()

    zrow = jnp.zeros((1, W, 2 * C), jnp.bfloat16)
    for ref in (xlc_ref, xr2_ref, hlc_ref, hr2_ref):
        ref[0:1] = zrow
        ref[H + 1:H + 2] = zrow
    xr2_ref[H:H + 1, :, C:2 * C] = zrow[:, :, 0:C]
    hr2_ref[H:H + 1, :, C:2 * C] = zrow[:, :, 0:C]

    def _fill(lc_ref, r2_ref, v):
        lc_ref[1:H + 1, :, 0:C] = _shift_cols_right(v)
        lc_ref[1:H + 1, :, C:2 * C] = v
        vr = _shift_cols_left(v)
        r2_ref[1:H + 1, :, 0:C] = vr
        r2_ref[0:H, :, C:2 * C] = vr

    def _conv(lc_ref, r2_ref, ws_ref, extra, emit):
        w0 = ws_ref[0]
        w1 = ws_ref[1]
        w2 = ws_ref[2]

        def lhs_slice(k, s):
            r0 = _TH * s
            if k == 0:
                blk = lc_ref[r0:r0 + _TH + 1]
            elif k == 1:
                blk = lc_ref[r0 + 1:r0 + _TH + 2]
            else:
                blk = r2_ref[r0:r0 + _TH + 1]
            return blk.reshape(MD, 2 * C)

        def strip(pair, mxu, half):
            return 4 * pair + 2 * mxu + half

        for pair in range(NP):
            for k, wk, sr, need_push in (
                    (0, w0, 0, True), (1, w1, 1, True),
                    (2, w2, 1, True)):
                for mxu in range(2):
                    if need_push:
                        pltpu.matmul_push_rhs(wk, staging_register=sr,
                                              mxu_index=mxu)
                for half in range(2):
                    for mxu in range(2):
                        pltpu.matmul_acc_lhs(
                            acc_addr=half * (MD // 4),
                            lhs=lhs_slice(k, strip(pair, mxu, half)),
                            mxu_index=mxu,
                            load_staged_rhs=sr if half == 0 else None)
            for half in range(2):
                for mxu in range(2):
                    s = strip(pair, mxu, half)
                    r = pltpu.matmul_pop(
                        acc_addr=half * (MD // 4), shape=(MD, 2 * C),
                        dtype=jnp.float32, mxu_index=mxu)
                    s0 = s * MS
                    y = (r[0:MS, 0:C] + r[W:MD, C:2 * C]
                         + extra[s0:s0 + MS])
                    emit(s, y)

    _fill(xlc_ref, xr2_ref, x_ref[...].astype(jnp.bfloat16))

    def emit1(s, y):
        h = jnp.maximum(y, 0.0).reshape(_TH, W, C).astype(jnp.bfloat16)
        r0 = s * _TH
        hlc_ref[1 + r0:1 + r0 + _TH, :, 0:C] = _shift_cols_right(h)
        hlc_ref[1 + r0:1 + r0 + _TH, :, C:2 * C] = h
        hr = _shift_cols_left(h)
        hr2_ref[1 + r0:1 + r0 + _TH, :, 0:C] = hr
        hr2_ref[r0:r0 + _TH, :, C:2 * C] = hr

    _conv(xlc_ref, xr2_ref, w1s_ref,
          jnp.broadcast_to(b1_ref[...], (M, C)), emit1)

    extra2 = b2_ref[...] + x_ref[...].astype(jnp.float32).reshape(M, C)

    def emit2(s, y):
        r0 = s * _TH
        out_ref[r0:r0 + _TH] = jnp.maximum(y, 0.0).reshape(
            _TH, W, C).astype(out_ref.dtype)

    _conv(hlc_ref, hr2_ref, w2s_ref, extra2, emit2)


def kernel(x_nhwc, w1f, bias1, w2f, bias2):
    N, H, W, C = x_nhwc.shape
    assert C % _LANE == 0 and W % 8 == 0 and H % (4 * _TH) == 0, \
        (N, H, W, C)

    b1 = bias1.astype(jnp.float32).reshape(1, C)
    b2 = bias2.astype(jnp.float32).reshape(1, C)

    def const_spec(shape):
        return pl.BlockSpec(shape, lambda n: tuple(0 for _ in shape),
                            pipeline_mode=pl.Buffered(1))

    return pl.pallas_call(
        _rb_kernel,
        out_shape=jax.ShapeDtypeStruct((N, H, W, C), x_nhwc.dtype),
        grid=(N,),
        in_specs=[
            pl.BlockSpec((None, H, W, C), lambda n: (n, 0, 0, 0)),
            const_spec((3, 3, C, C)),
            const_spec((3, 3, C, C)),
            const_spec((1, C)),
            const_spec((1, C)),
        ],
        out_specs=pl.BlockSpec((None, H, W, C), lambda n: (n, 0, 0, 0)),
        scratch_shapes=(
            [pltpu.VMEM((H + 2, W, 2 * C), jnp.bfloat16)
             for _ in range(4)]
            + [pltpu.VMEM((3, 2 * C, 2 * C), jnp.bfloat16)
               for _ in range(2)]),
        compiler_params=pltpu.CompilerParams(
            dimension_semantics=("parallel",)),
    )(x_nhwc, w1f.astype(jnp.bfloat16), w2f.astype(jnp.bfloat16), b1, b2)


# speedup vs baseline: 1.1236x; 1.0614x over previous
import jax
import jax.numpy as jnp
from jax.experimental import pallas as pl
from jax.experimental.pallas import tpu as pltpu

_LANE = 128
_TH = 7


def _shift_cols_right(a):
    zero = jnp.zeros_like(a[:, :1, :])
    return jnp.concatenate([zero, a[:, :-1, :]], axis=1)


def _shift_cols_left(a):
    zero = jnp.zeros_like(a[:, :1, :])
    return jnp.concatenate([a[:, 1:, :], zero], axis=1)


def _rb_kernel(x_ref, w1_ref, w2_ref, b1_ref, b2_ref, out_ref,
               xlc_ref, xr2_ref, hlc_ref, hr2_ref, w1s_ref, w2s_ref):
    H, W, C = x_ref.shape
    M = H * W
    MS = _TH * W
    MD = MS + W
    NS = H // _TH

    def _pack():
        zc = jnp.zeros((C, C), jnp.bfloat16)
        for ws, w in ((w1s_ref, w1_ref), (w2s_ref, w2_ref)):
            ws[0, 0:C, 0:C] = w[0, 0]
            ws[0, 0:C, C:2 * C] = w[1, 0]
            ws[0, C:2 * C, 0:C] = w[0, 1]
            ws[0, C:2 * C, C:2 * C] = w[1, 1]
            ws[1, 0:C, 0:C] = zc
            ws[1, 0:C, C:2 * C] = w[2, 0]
            ws[1, C:2 * C, 0:C] = zc
            ws[1, C:2 * C, C:2 * C] = w[2, 1]
            ws[2, 0:C, 0:C] = w[0, 2]
            ws[2, 0:C, C:2 * C] = w[1, 2]
            ws[2, C:2 * C, 0:C] = zc
            ws[2, C:2 * C, C:2 * C] = w[2, 2]
    _pack()

    zrow = jnp.zeros((1, W, 2 * C), jnp.bfloat16)
    for ref in (xlc_ref, xr2_ref, hlc_ref, hr2_ref):
        ref[0:1] = zrow
        ref[H + 1:H + 2] = zrow
    xr2_ref[H:H + 1, :, C:2 * C] = zrow[:, :, 0:C]
    hr2_ref[H:H + 1, :, C:2 * C] = zrow[:, :, 0:C]

    def _fill(lc_ref, r2_ref, v):
        lc_ref[1:H + 1, :, 0:C] = _shift_cols_right(v)
        lc_ref[1:H + 1, :, C:2 * C] = v
        vr = _shift_cols_left(v)
        r2_ref[1:H + 1, :, 0:C] = vr
        r2_ref[0:H, :, C:2 * C] = vr

    def _lhs(lc_ref, r2_ref, k, s):
        r0 = _TH * s
        if k == 0:
            blk = lc_ref[r0:r0 + _TH + 1]
        elif k == 1:
            blk = lc_ref[r0 + 1:r0 + _TH + 2]
        else:
            blk = r2_ref[r0:r0 + _TH + 1]
        return blk.reshape(MD, 2 * C)

    def _strip(lc_ref, r2_ref, ws_ref, region, s, extra, emit):
        mxu = s % 2
        for k, sr in ((0, 0), (1, 1), (2, 0)):
            pltpu.matmul_push_rhs(ws_ref[k], staging_register=sr,
                                  mxu_index=mxu)
            pltpu.matmul_acc_lhs(
                acc_addr=region, lhs=_lhs(lc_ref, r2_ref, k, s),
                mxu_index=mxu, load_staged_rhs=sr)
        r = pltpu.matmul_pop(acc_addr=region, shape=(MD, 2 * C),
                             dtype=jnp.float32, mxu_index=mxu)
        s0 = s * MS
        y = r[0:MS, 0:C] + r[W:MD, C:2 * C] + extra[s0:s0 + MS]
        emit(s, y)

    _fill(xlc_ref, xr2_ref, x_ref[...].astype(jnp.bfloat16))

    def emit1(s, y):
        h = jnp.maximum(y, 0.0).reshape(_TH, W, C).astype(jnp.bfloat16)
        r0 = s * _TH
        hlc_ref[1 + r0:1 + r0 + _TH, :, 0:C] = _shift_cols_right(h)
        hlc_ref[1 + r0:1 + r0 + _TH, :, C:2 * C] = h
        hr = _shift_cols_left(h)
        hr2_ref[1 + r0:1 + r0 + _TH, :, 0:C] = hr
        hr2_ref[r0:r0 + _TH, :, C:2 * C] = hr

    def emit2(s, y):
        r0 = s * _TH
        out_ref[r0:r0 + _TH] = jnp.maximum(y, 0.0).reshape(
            _TH, W, C).astype(out_ref.dtype)

    b1 = jnp.broadcast_to(b1_ref[...], (M, C))
    extra2 = b2_ref[...] + x_ref[...].astype(jnp.float32).reshape(M, C)

    def c1(s):
        _strip(xlc_ref, xr2_ref, w1s_ref, 0, s, b1, emit1)

    def c2(s):
        _strip(hlc_ref, hr2_ref, w2s_ref, MD // 4, s, extra2, emit2)

    c1(0)
    c1(1)
    for s in range(NS - 2):
        c1(s + 2)
        c2(s)
    c2(NS - 2)
    c2(NS - 1)


def kernel(x_nhwc, w1f, bias1, w2f, bias2):
    N, H, W, C = x_nhwc.shape
    assert C % _LANE == 0 and W % 8 == 0 and H % (4 * _TH) == 0, \
        (N, H, W, C)

    b1 = bias1.astype(jnp.float32).reshape(1, C)
    b2 = bias2.astype(jnp.float32).reshape(1, C)

    def const_spec(shape):
        return pl.BlockSpec(shape, lambda n: tuple(0 for _ in shape),
                            pipeline_mode=pl.Buffered(1))

    return pl.pallas_call(
        _rb_kernel,
        out_shape=jax.ShapeDtypeStruct((N, H, W, C), x_nhwc.dtype),
        grid=(N,),
        in_specs=[
            pl.BlockSpec((None, H, W, C), lambda n: (n, 0, 0, 0)),
            const_spec((3, 3, C, C)),
            const_spec((3, 3, C, C)),
            const_spec((1, C)),
            const_spec((1, C)),
        ],
        out_specs=pl.BlockSpec((None, H, W, C), lambda n: (n, 0, 0, 0)),
        scratch_shapes=(
            [pltpu.VMEM((H + 2, W, 2 * C), jnp.bfloat16)
             for _ in range(4)]
            + [pltpu.VMEM((3, 2 * C, 2 * C), jnp.bfloat16)
               for _ in range(2)]),
        compiler_params=pltpu.CompilerParams(
            dimension_semantics=("parallel",)),
    )(x_nhwc, w1f.astype(jnp.bfloat16), w2f.astype(jnp.bfloat16), b1, b2)
